# baseline (device time: 98584 ns/iter reference)
import os

import jax
import jax.numpy as jnp
from jax import lax
from jax.experimental import pallas as pl
from jax.experimental.pallas import tpu as pltpu

_VARIANT = os.environ.get("KVARIANT", "full")
_DO_COMM = _VARIANT != "compute"

N_DEV = 8
N_LOCAL_E = 8
N_TOK = 2048
D_MODEL = 512
D_FF = 1024
N_EXPERTS = 64
C = 320
TB = 512


def kernel(x, router_W, route_idx, expert_W, shared_W):
    def body(x_ref, rw_ref, idx_ref, ew_ref, sw_ref, out_ref,
             ag_send, ag_recv, ag_ssem, ag_rsem):
        my = lax.axis_index("i")

        barrier_sem = pltpu.get_barrier_semaphore()
        for d in range(1, N_DEV):
            pl.semaphore_signal(
                barrier_sem, inc=1,
                device_id=(lax.rem(my + d, N_DEV),),
                device_id_type=pl.DeviceIdType.MESH,
            )
        pl.semaphore_wait(barrier_sem, N_DEV - 1)

        scores = jnp.dot(x_ref[:, :], rw_ref[:, :],
                         preferred_element_type=jnp.float32)
        s_max = jnp.max(scores, axis=1, keepdims=True)
        ex = jnp.exp(scores - s_max)
        probs = ex / jnp.sum(ex, axis=1, keepdims=True)
        idx = idx_ref[:, 0:1]
        onehot = idx == lax.broadcasted_iota(jnp.int32, (1, N_EXPERTS), 1)
        p = jnp.sum(jnp.where(onehot, probs, 0.0), axis=1, keepdims=True)

        owner = lax.div(idx, N_LOCAL_E)
        local_ids = my * N_LOCAL_E + lax.broadcasted_iota(
            jnp.int32, (1, N_LOCAL_E), 1)
        wts = jnp.where(idx == local_ids, p, 0.0)

        oh_dev = (owner == lax.broadcasted_iota(
            jnp.int32, (1, N_DEV), 1)).astype(jnp.float32)
        r_i = lax.broadcasted_iota(jnp.int32, (TB, TB), 0)
        c_i = lax.broadcasted_iota(jnp.int32, (TB, TB), 1)
        l_strict = (r_i > c_i).astype(jnp.float32)
        off = jnp.zeros((1, N_DEV), jnp.float32)
        rank_blocks = []
        for b in range(N_TOK // TB):
            mb = oh_dev[b * TB:(b + 1) * TB, :]
            pref = jnp.dot(l_strict, mb,
                           preferred_element_type=jnp.float32) + off
            rank_blocks.append(
                jnp.sum(pref * mb, axis=1, keepdims=True))
            off = off + jnp.sum(mb, axis=0, keepdims=True)
        rank = jnp.concatenate(rank_blocks, axis=0)
        rank_i = rank.astype(jnp.int32)

        lane_c = lax.broadcasted_iota(jnp.int32, (1, C), 1)

        def sel_matrix(dev):
            return ((rank_i == lane_c) & (owner == dev)).astype(jnp.bfloat16)

        xbf = x_ref[:, :].astype(jnp.bfloat16)
        s_me = sel_matrix(my)
        dn_t = (((0,), (0,)), ((), ()))
        xg = lax.dot_general(
            s_me, xbf, dn_t,
            preferred_element_type=jnp.float32).astype(jnp.bfloat16)
        wg = lax.dot_general(s_me.astype(jnp.float32), wts, dn_t,
                             preferred_element_type=jnp.float32)
        yg = jnp.zeros((C, D_FF), jnp.float32)
        for j in range(N_LOCAL_E):
            xm = xg * wg[:, j:j + 1].astype(jnp.bfloat16)
            yg = yg + jnp.dot(xm, ew_ref[j].astype(jnp.bfloat16),
                              preferred_element_type=jnp.float32)
        yg_bf = yg.astype(jnp.bfloat16)
        ag_send[:, :] = yg_bf

        ag_rdmas = []
        for d in range(1, N_DEV):
            t = lax.rem(my + d, N_DEV)
            rdma = pltpu.make_async_remote_copy(
                src_ref=ag_send,
                dst_ref=ag_recv.at[d - 1],
                send_sem=ag_ssem.at[d - 1],
                recv_sem=ag_rsem.at[d - 1],
                device_id=(t,),
                device_id_type=pl.DeviceIdType.MESH,
            )
            if _DO_COMM:
                rdma.start()
            ag_rdmas.append(rdma)

        shared = jnp.dot(xbf, sw_ref[:, :].astype(jnp.bfloat16),
                         preferred_element_type=jnp.float32)
        out_ref[:, :] = shared + jnp.dot(
            s_me, yg_bf, preferred_element_type=jnp.float32)

        def slot_sender(j):
            return lax.rem(my + N_DEV - (j + 1), N_DEV)

        for j0 in (0, 2, 4):
            if _DO_COMM:
                ag_rdmas[j0].wait_recv()
                ag_rdmas[j0 + 1].wait_recv()
            s_pair = jnp.concatenate(
                [sel_matrix(slot_sender(j0)),
                 sel_matrix(slot_sender(j0 + 1))], axis=1)
            y_pair = jnp.concatenate(
                [ag_recv[j0], ag_recv[j0 + 1]], axis=0)
            out_ref[:, :] = out_ref[:, :] + jnp.dot(
                s_pair, y_pair, preferred_element_type=jnp.float32)

        if _DO_COMM:
            ag_rdmas[6].wait_recv()
        out_ref[:, :] = out_ref[:, :] + jnp.dot(
            sel_matrix(slot_sender(6)), ag_recv[6],
            preferred_element_type=jnp.float32)

        if _DO_COMM:
            for r in ag_rdmas:
                r.wait_send()

    return pl.pallas_call(
        body,
        out_shape=jax.ShapeDtypeStruct((N_TOK, D_FF), jnp.float32),
        in_specs=[pl.BlockSpec(memory_space=pltpu.VMEM)] * 5,
        out_specs=pl.BlockSpec(memory_space=pltpu.VMEM),
        scratch_shapes=[
            pltpu.VMEM((C, D_FF), jnp.bfloat16),
            pltpu.VMEM((N_DEV - 1, C, D_FF), jnp.bfloat16),
            pltpu.SemaphoreType.DMA((N_DEV - 1,)),
            pltpu.SemaphoreType.DMA((N_DEV - 1,)),
        ],
        compiler_params=pltpu.CompilerParams(
            collective_id=0,
            vmem_limit_bytes=120 * 1024 * 1024,
        ),
    )(x, router_W, route_idx, expert_W, shared_W)


# device time: 93852 ns/iter; 1.0504x vs baseline; 1.0504x over previous
import os

import jax
import jax.numpy as jnp
from jax import lax
from jax.experimental import pallas as pl
from jax.experimental.pallas import tpu as pltpu

_VARIANT = os.environ.get("KVARIANT", "full")
_DO_COMM = _VARIANT != "compute"

N_DEV = 8
N_LOCAL_E = 8
N_TOK = 2048
D_MODEL = 512
D_FF = 1024
N_EXPERTS = 64
C = 320
TB = 512


def kernel(x, router_W, route_idx, expert_W, shared_W):
    def body(x_ref, rw_ref, idx_ref, ew_ref, sw_ref, out_ref,
             ag_send, ag_recv, ag_ssem, ag_rsem):
        my = lax.axis_index("i")

        barrier_sem = pltpu.get_barrier_semaphore()
        for d in range(1, N_DEV):
            pl.semaphore_signal(
                barrier_sem, inc=1,
                device_id=(lax.rem(my + d, N_DEV),),
                device_id_type=pl.DeviceIdType.MESH,
            )
        pl.semaphore_wait(barrier_sem, N_DEV - 1)

        scores = jnp.dot(x_ref[:, :], rw_ref[:, :],
                         preferred_element_type=jnp.float32)
        s_max = jnp.max(scores, axis=1, keepdims=True)
        ex = jnp.exp(scores - s_max)
        probs = ex / jnp.sum(ex, axis=1, keepdims=True)
        idx = idx_ref[:, 0:1]
        onehot = idx == lax.broadcasted_iota(jnp.int32, (1, N_EXPERTS), 1)
        p = jnp.sum(jnp.where(onehot, probs, 0.0), axis=1, keepdims=True)

        owner = lax.div(idx, N_LOCAL_E)
        local_ids = my * N_LOCAL_E + lax.broadcasted_iota(
            jnp.int32, (1, N_LOCAL_E), 1)
        wts = jnp.where(idx == local_ids, p, 0.0)

        oh_dev = (owner == lax.broadcasted_iota(
            jnp.int32, (1, N_DEV), 1)).astype(jnp.float32)
        r_i = lax.broadcasted_iota(jnp.int32, (TB, TB), 0)
        c_i = lax.broadcasted_iota(jnp.int32, (TB, TB), 1)
        l_strict = (r_i > c_i).astype(jnp.float32)
        off = jnp.zeros((1, N_DEV), jnp.float32)
        rank_blocks = []
        for b in range(N_TOK // TB):
            mb = oh_dev[b * TB:(b + 1) * TB, :]
            pref = jnp.dot(l_strict, mb,
                           preferred_element_type=jnp.float32) + off
            rank_blocks.append(
                jnp.sum(pref * mb, axis=1, keepdims=True))
            off = off + jnp.sum(mb, axis=0, keepdims=True)
        rank = jnp.concatenate(rank_blocks, axis=0)
        rank_i = rank.astype(jnp.int32)

        lane_c = lax.broadcasted_iota(jnp.int32, (1, C), 1)

        def sel_matrix(dev):
            return ((rank_i == lane_c) & (owner == dev)).astype(jnp.bfloat16)

        xbf = x_ref[:, :].astype(jnp.bfloat16)
        s_me = sel_matrix(my)
        dn_t = (((0,), (0,)), ((), ()))
        xg = lax.dot_general(
            s_me, xbf, dn_t,
            preferred_element_type=jnp.float32).astype(jnp.bfloat16)
        wg = lax.dot_general(s_me.astype(jnp.float32), wts, dn_t,
                             preferred_element_type=jnp.float32)
        yg = jnp.zeros((C, D_FF), jnp.float32)
        for j in range(N_LOCAL_E):
            xm = xg * wg[:, j:j + 1].astype(jnp.bfloat16)
            yg = yg + jnp.dot(xm, ew_ref[j].astype(jnp.bfloat16),
                              preferred_element_type=jnp.float32)
        yg_bf = yg.astype(jnp.bfloat16)
        ag_send[:, :] = yg_bf

        ag_rdmas = []
        for d in range(1, N_DEV):
            t = lax.rem(my + d, N_DEV)
            rdma = pltpu.make_async_remote_copy(
                src_ref=ag_send,
                dst_ref=ag_recv.at[d - 1],
                send_sem=ag_ssem.at[d - 1],
                recv_sem=ag_rsem.at[d - 1],
                device_id=(t,),
                device_id_type=pl.DeviceIdType.MESH,
            )
            if _DO_COMM:
                rdma.start()
            ag_rdmas.append(rdma)

        shared = jnp.dot(xbf, sw_ref[:, :].astype(jnp.bfloat16),
                         preferred_element_type=jnp.float32)
        out_ref[:, :] = shared + jnp.dot(
            s_me, yg_bf, preferred_element_type=jnp.float32)

        for j in range(N_DEV - 1):
            if _DO_COMM:
                ag_rdmas[j].wait_recv()
            s = lax.rem(my + N_DEV - (j + 1), N_DEV)
            out_ref[:, :] = out_ref[:, :] + jnp.dot(
                sel_matrix(s), ag_recv[j],
                preferred_element_type=jnp.float32)

        if _DO_COMM:
            for r in ag_rdmas:
                r.wait_send()

    return pl.pallas_call(
        body,
        out_shape=jax.ShapeDtypeStruct((N_TOK, D_FF), jnp.float32),
        in_specs=[pl.BlockSpec(memory_space=pltpu.VMEM)] * 5,
        out_specs=pl.BlockSpec(memory_space=pltpu.VMEM),
        scratch_shapes=[
            pltpu.VMEM((C, D_FF), jnp.bfloat16),
            pltpu.VMEM((N_DEV - 1, C, D_FF), jnp.bfloat16),
            pltpu.SemaphoreType.DMA((N_DEV - 1,)),
            pltpu.SemaphoreType.DMA((N_DEV - 1,)),
        ],
        compiler_params=pltpu.CompilerParams(
            collective_id=0,
            vmem_limit_bytes=120 * 1024 * 1024,
        ),
    )(x, router_W, route_idx, expert_W, shared_W)
